# Initial kernel scaffold; baseline (speedup 1.0000x reference)
import jax
import jax.numpy as jnp
from jax.experimental import pallas as pl


def kernel(letters, table):
    raise NotImplementedError("write your pallas kernel here")


# speedup vs baseline: 3.8666x; 3.8666x over previous
import jax
import jax.numpy as jnp
from jax import lax
from jax.experimental import pallas as pl
from jax.experimental.pallas import tpu as pltpu
from jax.experimental.pallas import tpu_sc as plsc

DIM = 25
BATCH = 16384
SEQ = 200
TOTAL = BATCH * SEQ
NC, NS = 2, 16
NW = NC * NS
PER_W = TOTAL // NW
CHUNK = 4096
STEPS = PER_W // CHUNK
GROUPS = CHUNK // 16
TBL = 34 * DIM


def _body(letters_hbm, table_hbm, out_hbm, table_v, idx_v, rows_v, sem):
    wid = lax.axis_index("s") * NC + lax.axis_index("c")
    base_w = wid * PER_W
    pltpu.sync_copy(table_hbm, table_v)
    lane25 = lax.iota(jnp.int32, 16) * DIM

    def step(i, carry):
        base = base_w + i * CHUNK
        pltpu.sync_copy(letters_hbm.at[pl.ds(base, CHUNK)], idx_v)

        def grp(g, c2):
            lvec = idx_v[pl.ds(g * 16, 16)]
            src0 = lvec * DIM
            dst0 = lane25 + g * (16 * DIM)
            for d in range(DIM):
                vals = plsc.load_gather(table_v, [src0 + d])
                plsc.store_scatter(rows_v, [dst0 + d], vals)
            return c2

        lax.fori_loop(0, GROUPS, grp, 0)
        pltpu.sync_copy(rows_v, out_hbm.at[pl.ds(base * DIM, CHUNK * DIM)])
        return carry

    lax.fori_loop(0, STEPS, step, 0)


@jax.jit
def _gather(letters_flat, table_flat):
    mesh = plsc.VectorSubcoreMesh(core_axis_name="c", subcore_axis_name="s")
    k = pl.kernel(
        _body,
        out_type=jax.ShapeDtypeStruct((TOTAL * DIM,), jnp.float32),
        mesh=mesh,
        scratch_types=[
            pltpu.VMEM((TBL,), jnp.float32),
            pltpu.VMEM((CHUNK,), jnp.int32),
            pltpu.VMEM((CHUNK * DIM,), jnp.float32),
            pltpu.SemaphoreType.DMA,
        ],
        compiler_params=pltpu.CompilerParams(
            use_tc_tiling_on_sc=False, needs_layout_passes=False
        ),
    )
    return k(letters_flat, table_flat)


def kernel(letters, table):
    lf = letters.reshape(TOTAL).astype(jnp.int32)
    out = _gather(lf, table.reshape(TBL))
    return out.reshape(BATCH, SEQ, DIM)


# speedup vs baseline: 16.1268x; 4.1708x over previous
import jax
import jax.numpy as jnp
from jax import lax
from jax.experimental import pallas as pl
from jax.experimental.pallas import tpu as pltpu
from jax.experimental.pallas import tpu_sc as plsc

DIM = 25
BATCH = 16384
SEQ = 200
NC, NS = 2, 16
NW = NC * NS
W = BATCH // NW
ROWS = SEQ // 8
TBL = 34 * DIM


def _body(letters_hbm, table_hbm, out_hbm, table_v, idx_v, rows_v, sem):
    wid = lax.axis_index("s") * NC + lax.axis_index("c")
    lane0 = wid * W
    pltpu.sync_copy(table_hbm, table_v)

    def step(j, carry):
        pltpu.sync_copy(letters_hbm.at[pl.ds(j * 8, 8), pl.ds(lane0, W)], idx_v)

        for ss in range(8):
            def grp(k, c2):
                lvec = idx_v[ss, pl.ds(k * 16, 16)]
                src0 = lvec * DIM
                for d in range(DIM):
                    rows_v[d, ss, pl.ds(k * 16, 16)] = plsc.load_gather(
                        table_v, [src0 + d]
                    )
                return c2

            lax.fori_loop(0, W // 16, grp, 0)

        copies = [
            pltpu.async_copy(
                rows_v.at[d],
                out_hbm.at[d, pl.ds(j * 8, 8), pl.ds(lane0, W)],
                sem,
            )
            for d in range(DIM)
        ]
        for c in copies:
            c.wait()
        return carry

    lax.fori_loop(0, ROWS, step, 0)


@jax.jit
def _gather(letters_t, table_flat):
    mesh = plsc.VectorSubcoreMesh(core_axis_name="c", subcore_axis_name="s")
    k = pl.kernel(
        _body,
        out_type=jax.ShapeDtypeStruct((DIM, SEQ, BATCH), jnp.float32),
        mesh=mesh,
        scratch_types=[
            pltpu.VMEM((TBL,), jnp.float32),
            pltpu.VMEM((8, W), jnp.int32),
            pltpu.VMEM((DIM, 8, W), jnp.float32),
            pltpu.SemaphoreType.DMA,
        ],
        compiler_params=pltpu.CompilerParams(
            use_tc_tiling_on_sc=True, needs_layout_passes=False
        ),
    )
    return k(letters_t, table_flat)


def kernel(letters, table):
    out_t = _gather(letters.astype(jnp.int32).T, table.reshape(TBL))
    return out_t.transpose(2, 1, 0)


# speedup vs baseline: 37.5627x; 2.3292x over previous
import jax
import jax.numpy as jnp
from jax import lax
from jax.experimental import pallas as pl
from jax.experimental.pallas import tpu as pltpu
from jax.experimental.pallas import tpu_sc as plsc

DIM = 25
BATCH = 16384
SEQ = 200
NC, NS = 2, 16
NW = NC * NS
W = BATCH // NW
ROWS = SEQ // 8
TBL = 34 * DIM


def _body(letters_hbm, table_hbm, out_hbm, table_v, idx_v, rows_v, sem):
    wid = lax.axis_index("s") * NC + lax.axis_index("c")
    lane0 = wid * W
    pltpu.sync_copy(table_hbm, table_v)

    def step(j, carry):
        pltpu.sync_copy(letters_hbm.at[pl.ds(j * 8, 8), pl.ds(lane0, W)], idx_v)

        for ss in range(8):
            @plsc.parallel_loop(0, W // 16, unroll=2)
            def grp(k):
                lvec = idx_v[ss, pl.ds(k * 16, 16)]
                src0 = lvec * DIM
                for d in range(DIM):
                    rows_v[d, ss, pl.ds(k * 16, 16)] = plsc.load_gather(
                        table_v, [src0 + d]
                    )

        copies = [
            pltpu.async_copy(
                rows_v.at[d],
                out_hbm.at[d, pl.ds(j * 8, 8), pl.ds(lane0, W)],
                sem,
            )
            for d in range(DIM)
        ]
        for c in copies:
            c.wait()
        return carry

    lax.fori_loop(0, ROWS, step, 0)


@jax.jit
def _gather(letters_t, table_flat):
    mesh = plsc.VectorSubcoreMesh(core_axis_name="c", subcore_axis_name="s")
    k = pl.kernel(
        _body,
        out_type=jax.ShapeDtypeStruct((DIM, SEQ, BATCH), jnp.float32),
        mesh=mesh,
        scratch_types=[
            pltpu.VMEM((TBL,), jnp.float32),
            pltpu.VMEM((8, W), jnp.int32),
            pltpu.VMEM((DIM, 8, W), jnp.float32),
            pltpu.SemaphoreType.DMA,
        ],
        compiler_params=pltpu.CompilerParams(
            use_tc_tiling_on_sc=True, needs_layout_passes=False
        ),
    )
    return k(letters_t, table_flat)


def kernel(letters, table):
    out_t = _gather(letters.astype(jnp.int32).T, table.reshape(TBL))
    return out_t.transpose(2, 1, 0)


# speedup vs baseline: 40.1309x; 1.0684x over previous
import jax
import jax.numpy as jnp
from jax import lax
from jax.experimental import pallas as pl
from jax.experimental.pallas import tpu as pltpu
from jax.experimental.pallas import tpu_sc as plsc

DIM = 25
BATCH = 16384
SEQ = 200
NC, NS = 2, 16
NW = NC * NS
W = BATCH // NW
W2 = W // 2
ROWS = SEQ // 8
TBL = 34 * DIM


def _body(letters_hbm, table_hbm, out_hbm, table_v, idx2, rows_v,
          sem_in, sem_o0, sem_o1):
    wid = lax.axis_index("s") * NC + lax.axis_index("c")
    lane0 = wid * W
    pltpu.sync_copy(table_hbm, table_v)
    sems = (sem_o0, sem_o1)

    def idx_copy(j, buf):
        return pltpu.make_async_copy(
            letters_hbm.at[pl.ds(j * 8, 8), pl.ds(lane0, W)],
            idx2.at[buf], sem_in,
        )

    def half_out_copy(j, h, d):
        return pltpu.make_async_copy(
            rows_v.at[d, :, pl.ds(h * W2, W2)],
            out_hbm.at[d, pl.ds(j * 8, 8), pl.ds(lane0 + h * W2, W2)],
            sems[h],
        )

    def half_drain(j, h):
        pltpu.make_async_copy(
            rows_v.at[:, :, pl.ds(h * W2, W2)],
            out_hbm.at[:, pl.ds(j * 8, 8), pl.ds(lane0 + h * W2, W2)],
            sems[h],
        ).wait()

    idx_copy(0, 0).start()

    def step(j, carry):
        buf = lax.rem(j, 2)
        idx_copy(j, buf).wait()

        @pl.when(j + 1 < ROWS)
        def _():
            idx_copy(j + 1, lax.rem(j + 1, 2)).start()

        for h in range(2):
            @pl.when(j > 0)
            def _():
                half_drain(j - 1, h)

            for ss in range(8):
                @plsc.parallel_loop(0, W2 // 16, unroll=2)
                def grp(k):
                    col = h * W2 + k * 16
                    lvec = idx2[buf, ss, pl.ds(col, 16)]
                    src0 = lvec * DIM
                    for d in range(DIM):
                        rows_v[d, ss, pl.ds(col, 16)] = plsc.load_gather(
                            table_v, [src0 + d]
                        )

            for d in range(DIM):
                half_out_copy(j, h, d).start()
        return carry

    lax.fori_loop(0, ROWS, step, 0)
    for h in range(2):
        half_drain(ROWS - 1, h)


@jax.jit
def _gather(letters_t, table_flat):
    mesh = plsc.VectorSubcoreMesh(core_axis_name="c", subcore_axis_name="s")
    k = pl.kernel(
        _body,
        out_type=jax.ShapeDtypeStruct((DIM, SEQ, BATCH), jnp.float32),
        mesh=mesh,
        scratch_types=[
            pltpu.VMEM((TBL,), jnp.float32),
            pltpu.VMEM((2, 8, W), jnp.int32),
            pltpu.VMEM((DIM, 8, W), jnp.float32),
            pltpu.SemaphoreType.DMA,
            pltpu.SemaphoreType.DMA,
            pltpu.SemaphoreType.DMA,
        ],
        compiler_params=pltpu.CompilerParams(
            use_tc_tiling_on_sc=True, needs_layout_passes=False
        ),
    )
    return k(letters_t, table_flat)


def kernel(letters, table):
    out_t = _gather(letters.astype(jnp.int32).T, table.reshape(TBL))
    return out_t.transpose(2, 1, 0)
